# Initial kernel scaffold; baseline (speedup 1.0000x reference)
import jax
import jax.numpy as jnp
from jax.experimental import pallas as pl


def kernel(uid, user_gender, user_age, user_job, uid_table, gender_table, age_table, job_table, Wu, bu, Wg, bg, Wa, ba, Wj, bj, Wc, bc):
    raise NotImplementedError("write your pallas kernel here")


# speedup vs baseline: 2.7829x; 2.7829x over previous
import functools

import jax
import jax.numpy as jnp
from jax import lax
from jax.experimental import pallas as pl
from jax.experimental.pallas import tpu as pltpu
from jax.experimental.pallas import tpu_sc as plsc

B = 4096
D = 128
DH = 64
OUT = 200
R = 512
G = B // R
NCLS = 40


def _sc_gather(table, idx):
  info = plsc.get_sparse_core_info()
  nc, ns = info.num_cores, info.num_subcores
  nw = nc * ns
  b_per_w = B // nw
  mesh = plsc.VectorSubcoreMesh(core_axis_name="c", subcore_axis_name="s")

  @functools.partial(
      pl.kernel,
      mesh=mesh,
      out_type=jax.ShapeDtypeStruct((B, D), jnp.float32),
      scratch_types=[
          pltpu.VMEM((b_per_w,), jnp.int32),
          pltpu.VMEM((b_per_w, D), jnp.float32),
          pltpu.SemaphoreType.DMA,
      ],
  )
  def gather_kernel(table_hbm, idx_hbm, out_hbm, idx_v, rows_v, sem):
    wid = lax.axis_index("s") * nc + lax.axis_index("c")
    base = wid * b_per_w
    pltpu.sync_copy(idx_hbm.at[pl.ds(base, b_per_w)], idx_v)
    pltpu.async_copy(table_hbm.at[idx_v], rows_v, sem).wait()
    pltpu.sync_copy(rows_v, out_hbm.at[pl.ds(base, b_per_w)])

  return gather_kernel(table, idx)


def _tc_body(uid_rows_ref, idx_ref, g8_ref, a8_ref, j24_ref,
             Wu_ref, bu_ref, Wg_ref, bg_ref, Wa_ref, ba_ref, Wj_ref, bj_ref,
             Wc_ref, bc_ref, out_ref, ctr_ref):
  f32 = jnp.float32
  i = pl.program_id(0)

  @pl.when(i == 0)
  def _():
    gt = jnp.maximum(
        jnp.dot(g8_ref[...], Wg_ref[...], preferred_element_type=f32)
        + bg_ref[...], 0.0)
    at = jnp.maximum(
        jnp.dot(a8_ref[...], Wa_ref[...], preferred_element_type=f32)
        + ba_ref[...], 0.0)
    jt = jnp.maximum(
        jnp.dot(j24_ref[...], Wj_ref[...], preferred_element_type=f32)
        + bj_ref[...], 0.0)
    ctr_ref[0:8, :] = jnp.dot(gt, Wc_ref[D:2 * D, :],
                              preferred_element_type=f32)
    ctr_ref[8:16, :] = jnp.dot(at, Wc_ref[2 * D:3 * D, :],
                               preferred_element_type=f32)
    ctr_ref[16:40, :] = jnp.dot(jt, Wc_ref[3 * D:4 * D, :],
                                preferred_element_type=f32)

  u_fc = jnp.maximum(
      jnp.dot(uid_rows_ref[...], Wu_ref[...], preferred_element_type=f32)
      + bu_ref[...], 0.0)
  acc = jnp.dot(u_fc, Wc_ref[0:D, :], preferred_element_type=f32)

  i3 = idx_ref[0]
  cls = lax.broadcasted_iota(jnp.int32, (R, NCLS), 1)
  oh = ((i3[:, 0:1] == cls) | (i3[:, 1:2] == cls)
        | (i3[:, 2:3] == cls)).astype(f32)
  acc = acc + jnp.dot(oh, ctr_ref[...], preferred_element_type=f32)
  acc = acc + bc_ref[...]

  t = jnp.tanh(acc)
  ssum = jnp.sum(t * t, axis=1, keepdims=True)
  norm = jnp.maximum(jnp.sqrt(ssum), 1e-12)
  out_ref[...] = t / norm


def _full(shape):
  return pl.BlockSpec(shape, lambda i: tuple(0 for _ in shape))


def kernel(uid, user_gender, user_age, user_job, uid_table, gender_table,
           age_table, job_table, Wu, bu, Wg, bg, Wa, ba, Wj, bj, Wc, bc):
  uid_rows = _sc_gather(uid_table, jnp.asarray(uid, jnp.int32))

  idx3 = jnp.stack([
      jnp.asarray(user_gender, jnp.int32),
      jnp.asarray(user_age, jnp.int32) + 8,
      jnp.asarray(user_job, jnp.int32) + 16,
  ], axis=-1).reshape(G, R, 3)

  g8 = jnp.zeros((8, DH), jnp.float32).at[0:2].set(gender_table)
  a8 = jnp.zeros((8, DH), jnp.float32).at[0:7].set(age_table)
  j24 = jnp.zeros((24, DH), jnp.float32).at[0:21].set(job_table)

  out = pl.pallas_call(
      _tc_body,
      grid=(G,),
      in_specs=[
          pl.BlockSpec((R, D), lambda i: (i, 0)),
          pl.BlockSpec((1, R, 3), lambda i: (i, 0, 0)),
          _full((8, DH)),
          _full((8, DH)),
          _full((24, DH)),
          _full((D, D)),
          _full((1, D)),
          _full((DH, D)),
          _full((1, D)),
          _full((DH, D)),
          _full((1, D)),
          _full((DH, D)),
          _full((1, D)),
          _full((4 * D, OUT)),
          _full((1, OUT)),
      ],
      out_specs=pl.BlockSpec((R, OUT), lambda i: (i, 0)),
      out_shape=jax.ShapeDtypeStruct((B, OUT), jnp.float32),
      scratch_shapes=[pltpu.VMEM((NCLS, OUT), jnp.float32)],
  )(uid_rows, idx3, g8, a8, j24,
    Wu, bu.reshape(1, D), Wg, bg.reshape(1, D), Wa, ba.reshape(1, D),
    Wj, bj.reshape(1, D), Wc, bc.reshape(1, OUT))
  return out


# speedup vs baseline: 2.8656x; 1.0297x over previous
import functools

import jax
import jax.numpy as jnp
from jax import lax
from jax.experimental import pallas as pl
from jax.experimental.pallas import tpu as pltpu
from jax.experimental.pallas import tpu_sc as plsc

B = 4096
D = 128
DH = 64
OUT = 200
R = 512
G = B // R
NCLS = 40
GENDER_N = 2
AGE_N = 7
JOB_N = 21


def _sc_gather(table, idx):
  info = plsc.get_sparse_core_info()
  nc, ns = info.num_cores, info.num_subcores
  nw = nc * ns
  b_per_w = B // nw
  mesh = plsc.VectorSubcoreMesh(core_axis_name="c", subcore_axis_name="s")

  @functools.partial(
      pl.kernel,
      mesh=mesh,
      out_type=jax.ShapeDtypeStruct((B, D), jnp.float32),
      scratch_types=[
          pltpu.VMEM((b_per_w,), jnp.int32),
          pltpu.VMEM((b_per_w, D), jnp.float32),
          pltpu.SemaphoreType.DMA,
      ],
  )
  def gather_kernel(table_hbm, idx_hbm, out_hbm, idx_v, rows_v, sem):
    wid = lax.axis_index("s") * nc + lax.axis_index("c")
    base = wid * b_per_w
    pltpu.sync_copy(idx_hbm.at[pl.ds(base, b_per_w)], idx_v)
    pltpu.async_copy(table_hbm.at[idx_v], rows_v, sem).wait()
    pltpu.sync_copy(rows_v, out_hbm.at[pl.ds(base, b_per_w)])

  return gather_kernel(table, idx)


def _tc_body(uid_rows_ref, gi_ref, ai_ref, ji_ref, g_tab_ref, a_tab_ref,
             j_tab_ref, Wu_ref, bu_ref, Wg_ref, bg_ref, Wa_ref, ba_ref,
             Wj_ref, bj_ref, Wc_ref, bc_ref, out_ref, ctr_ref):
  f32 = jnp.float32
  i = pl.program_id(0)

  @pl.when(i == 0)
  def _():
    ctr_ref[...] = jnp.zeros((NCLS, OUT), f32)
    gt = jnp.maximum(
        jnp.dot(g_tab_ref[...], Wg_ref[...], preferred_element_type=f32)
        + bg_ref[...], 0.0)
    at = jnp.maximum(
        jnp.dot(a_tab_ref[...], Wa_ref[...], preferred_element_type=f32)
        + ba_ref[...], 0.0)
    jt = jnp.maximum(
        jnp.dot(j_tab_ref[...], Wj_ref[...], preferred_element_type=f32)
        + bj_ref[...], 0.0)
    ctr_ref[0:2, :] = jnp.dot(gt, Wc_ref[D:2 * D, :],
                              preferred_element_type=f32)
    ctr_ref[8:15, :] = jnp.dot(at, Wc_ref[2 * D:3 * D, :],
                               preferred_element_type=f32)
    ctr_ref[16:37, :] = jnp.dot(jt, Wc_ref[3 * D:4 * D, :],
                                preferred_element_type=f32)

  u_fc = jnp.maximum(
      jnp.dot(uid_rows_ref[...], Wu_ref[...], preferred_element_type=f32)
      + bu_ref[...], 0.0)
  acc = jnp.dot(u_fc, Wc_ref[0:D, :], preferred_element_type=f32)

  cls = lax.broadcasted_iota(jnp.int32, (R, NCLS), 1)
  gi = gi_ref[0].reshape(R, 1)
  ai = ai_ref[0].reshape(R, 1)
  ji = ji_ref[0].reshape(R, 1)
  oh = ((gi == cls) | (ai == cls - 8) | (ji == cls - 16)).astype(f32)
  acc = acc + jnp.dot(oh, ctr_ref[...], preferred_element_type=f32)
  acc = acc + bc_ref[...]

  t = jnp.tanh(acc)
  ssum = jnp.sum(t * t, axis=1, keepdims=True)
  norm = jnp.maximum(jnp.sqrt(ssum), 1e-12)
  out_ref[...] = t / norm


def _full(shape):
  return pl.BlockSpec(shape, lambda i: tuple(0 for _ in shape))


def kernel(uid, user_gender, user_age, user_job, uid_table, gender_table,
           age_table, job_table, Wu, bu, Wg, bg, Wa, ba, Wj, bj, Wc, bc):
  uid_rows = _sc_gather(uid_table, jnp.asarray(uid, jnp.int32))

  gi = jnp.asarray(user_gender, jnp.int32).reshape(G, 1, R)
  ai = jnp.asarray(user_age, jnp.int32).reshape(G, 1, R)
  ji = jnp.asarray(user_job, jnp.int32).reshape(G, 1, R)

  idx_spec = pl.BlockSpec((1, 1, R), lambda i: (i, 0, 0))
  out = pl.pallas_call(
      _tc_body,
      grid=(G,),
      in_specs=[
          pl.BlockSpec((R, D), lambda i: (i, 0)),
          idx_spec,
          idx_spec,
          idx_spec,
          _full((GENDER_N, DH)),
          _full((AGE_N, DH)),
          _full((JOB_N, DH)),
          _full((D, D)),
          _full((1, D)),
          _full((DH, D)),
          _full((1, D)),
          _full((DH, D)),
          _full((1, D)),
          _full((DH, D)),
          _full((1, D)),
          _full((4 * D, OUT)),
          _full((1, OUT)),
      ],
      out_specs=pl.BlockSpec((R, OUT), lambda i: (i, 0)),
      out_shape=jax.ShapeDtypeStruct((B, OUT), jnp.float32),
      scratch_shapes=[pltpu.VMEM((NCLS, OUT), jnp.float32)],
  )(uid_rows, gi, ai, ji, gender_table, age_table, job_table,
    Wu, bu.reshape(1, D), Wg, bg.reshape(1, D), Wa, ba.reshape(1, D),
    Wj, bj.reshape(1, D), Wc, bc.reshape(1, OUT))
  return out


# speedup vs baseline: 3.3194x; 1.1584x over previous
import functools

import jax
import jax.numpy as jnp
from jax import lax
from jax.experimental import pallas as pl
from jax.experimental.pallas import tpu as pltpu
from jax.experimental.pallas import tpu_sc as plsc

B = 4096
D = 128
DH = 64
OUT = 200
R = 512
G = B // R
NCLS = 40
GENDER_N = 2
AGE_N = 7
JOB_N = 21


def _sc_gather(table, idx):
  info = plsc.get_sparse_core_info()
  nc, ns = info.num_cores, info.num_subcores
  nw = nc * ns
  b_per_w = B // nw
  mesh = plsc.VectorSubcoreMesh(core_axis_name="c", subcore_axis_name="s")

  @functools.partial(
      pl.kernel,
      mesh=mesh,
      out_type=jax.ShapeDtypeStruct((B, D), jnp.float32),
      scratch_types=[
          pltpu.VMEM((b_per_w,), jnp.int32),
          pltpu.VMEM((b_per_w, D), jnp.float32),
          pltpu.SemaphoreType.DMA,
      ],
  )
  def gather_kernel(table_hbm, idx_hbm, out_hbm, idx_v, rows_v, sem):
    wid = lax.axis_index("s") * nc + lax.axis_index("c")
    base = wid * b_per_w
    pltpu.sync_copy(idx_hbm.at[pl.ds(base, b_per_w)], idx_v)
    pltpu.async_copy(table_hbm.at[idx_v], rows_v, sem).wait()
    pltpu.sync_copy(rows_v, out_hbm.at[pl.ds(base, b_per_w)])

  return gather_kernel(table, idx)


def _nt(a, b):
  return lax.dot_general(a, b, (((1,), (1,)), ((), ())),
                         preferred_element_type=jnp.float32)


def _tc_body(uid_rows_ref, gi_ref, ai_ref, ji_ref, g_tab_ref, a_tab_ref,
             j_tab_ref, Wu_ref, bu_ref, Wg_ref, bg_ref, Wa_ref, ba_ref,
             Wj_ref, bj_ref, WcT_ref, bcT_ref, out_ref, ctrT_ref):
  f32 = jnp.float32
  i = pl.program_id(0)

  @pl.when(i == 0)
  def _():
    ctrT_ref[...] = jnp.zeros((OUT, NCLS), f32)
    gt = jnp.maximum(
        jnp.dot(g_tab_ref[...], Wg_ref[...], preferred_element_type=f32)
        + bg_ref[...], 0.0)
    at = jnp.maximum(
        jnp.dot(a_tab_ref[...], Wa_ref[...], preferred_element_type=f32)
        + ba_ref[...], 0.0)
    jt = jnp.maximum(
        jnp.dot(j_tab_ref[...], Wj_ref[...], preferred_element_type=f32)
        + bj_ref[...], 0.0)
    ctrT_ref[:, 0:2] = _nt(WcT_ref[:, D:2 * D], gt)
    ctrT_ref[:, 8:15] = _nt(WcT_ref[:, 2 * D:3 * D], at)
    ctrT_ref[:, 16:37] = _nt(WcT_ref[:, 3 * D:4 * D], jt)

  u_fc = jnp.maximum(
      jnp.dot(uid_rows_ref[...], Wu_ref[...], preferred_element_type=f32)
      + bu_ref[...], 0.0)
  accT = _nt(WcT_ref[:, 0:D], u_fc)

  cls = lax.broadcasted_iota(jnp.int32, (R, NCLS), 1)
  gi = gi_ref[0].reshape(R, 1)
  ai = ai_ref[0].reshape(R, 1)
  ji = ji_ref[0].reshape(R, 1)
  oh = ((gi == cls) | (ai == cls - 8) | (ji == cls - 16)).astype(f32)
  accT = accT + _nt(ctrT_ref[...], oh)
  accT = accT + bcT_ref[...]

  t = jnp.tanh(accT)
  ssum = jnp.sum(t * t, axis=0, keepdims=True)
  norm = jnp.maximum(jnp.sqrt(ssum), 1e-12)
  out_ref[...] = t / norm


def _full(shape):
  return pl.BlockSpec(shape, lambda i: tuple(0 for _ in shape))


def kernel(uid, user_gender, user_age, user_job, uid_table, gender_table,
           age_table, job_table, Wu, bu, Wg, bg, Wa, ba, Wj, bj, Wc, bc):
  uid_rows = _sc_gather(uid_table, jnp.asarray(uid, jnp.int32))

  gi = jnp.asarray(user_gender, jnp.int32).reshape(G, 1, R)
  ai = jnp.asarray(user_age, jnp.int32).reshape(G, 1, R)
  ji = jnp.asarray(user_job, jnp.int32).reshape(G, 1, R)

  idx_spec = pl.BlockSpec((1, 1, R), lambda i: (i, 0, 0))
  out = pl.pallas_call(
      _tc_body,
      grid=(G,),
      in_specs=[
          pl.BlockSpec((R, D), lambda i: (i, 0)),
          idx_spec,
          idx_spec,
          idx_spec,
          _full((GENDER_N, DH)),
          _full((AGE_N, DH)),
          _full((JOB_N, DH)),
          _full((D, D)),
          _full((1, D)),
          _full((DH, D)),
          _full((1, D)),
          _full((DH, D)),
          _full((1, D)),
          _full((DH, D)),
          _full((1, D)),
          _full((OUT, 4 * D)),
          _full((OUT, 1)),
      ],
      out_specs=pl.BlockSpec((OUT, R), lambda i: (0, i)),
      out_shape=jax.ShapeDtypeStruct((OUT, B), jnp.float32),
      scratch_shapes=[pltpu.VMEM((OUT, NCLS), jnp.float32)],
  )(uid_rows, gi, ai, ji, gender_table, age_table, job_table,
    Wu, bu.reshape(1, D), Wg, bg.reshape(1, D), Wa, ba.reshape(1, D),
    Wj, bj.reshape(1, D), Wc.T, bc.reshape(OUT, 1))
  return out.T


# speedup vs baseline: 3.6143x; 1.0888x over previous
import functools

import jax
import jax.numpy as jnp
from jax import lax
from jax.experimental import pallas as pl
from jax.experimental.pallas import tpu as pltpu
from jax.experimental.pallas import tpu_sc as plsc

B = 4096
D = 128
DH = 64
OUT = 200
R = 1024
G = B // R
NCLS = 40
GENDER_N = 2
AGE_N = 7
JOB_N = 21


def _sc_gather(table, idx):
  info = plsc.get_sparse_core_info()
  nc, ns = info.num_cores, info.num_subcores
  nw = nc * ns
  b_per_w = B // nw
  mesh = plsc.VectorSubcoreMesh(core_axis_name="c", subcore_axis_name="s")

  @functools.partial(
      pl.kernel,
      mesh=mesh,
      out_type=jax.ShapeDtypeStruct((B, D), jnp.float32),
      scratch_types=[
          pltpu.VMEM((b_per_w,), jnp.int32),
          pltpu.VMEM((b_per_w, D), jnp.float32),
          pltpu.SemaphoreType.DMA,
      ],
  )
  def gather_kernel(table_hbm, idx_hbm, out_hbm, idx_v, rows_v, sem):
    wid = lax.axis_index("s") * nc + lax.axis_index("c")
    base = wid * b_per_w
    pltpu.sync_copy(idx_hbm.at[pl.ds(base, b_per_w)], idx_v)
    pltpu.async_copy(table_hbm.at[idx_v], rows_v, sem).wait()
    pltpu.sync_copy(rows_v, out_hbm.at[pl.ds(base, b_per_w)])

  return gather_kernel(table, idx)


def _nt(a, b):
  return lax.dot_general(a, b, (((1,), (1,)), ((), ())),
                         preferred_element_type=jnp.float32)


def _tc_body(uid_rows_ref, gi_ref, ai_ref, ji_ref, g_tab_ref, a_tab_ref,
             j_tab_ref, Wu_ref, bu_ref, Wg_ref, bg_ref, Wa_ref, ba_ref,
             Wj_ref, bj_ref, WcT_ref, bc_ref, out_ref, ctrT_ref):
  f32 = jnp.float32
  i = pl.program_id(0)

  @pl.when(i == 0)
  def _():
    ctrT_ref[...] = jnp.zeros((OUT, NCLS), f32)
    gt = jnp.maximum(
        jnp.dot(g_tab_ref[...], Wg_ref[...], preferred_element_type=f32)
        + bg_ref[...], 0.0)
    at = jnp.maximum(
        jnp.dot(a_tab_ref[...], Wa_ref[...], preferred_element_type=f32)
        + ba_ref[...], 0.0)
    jt = jnp.maximum(
        jnp.dot(j_tab_ref[...], Wj_ref[...], preferred_element_type=f32)
        + bj_ref[...], 0.0)
    bcT = bc_ref[...].reshape(OUT, 1)
    ctrT_ref[:, 0:2] = _nt(WcT_ref[:, D:2 * D], gt) + bcT
    ctrT_ref[:, 8:15] = _nt(WcT_ref[:, 2 * D:3 * D], at)
    ctrT_ref[:, 16:37] = _nt(WcT_ref[:, 3 * D:4 * D], jt)

  u_fc = jnp.maximum(
      jnp.dot(uid_rows_ref[...], Wu_ref[...], preferred_element_type=f32)
      + bu_ref[...], 0.0)
  accT = _nt(WcT_ref[:, 0:D], u_fc)

  cls = lax.broadcasted_iota(jnp.int32, (R, NCLS), 1)
  gi = gi_ref[0].reshape(R, 1)
  ai = ai_ref[0].reshape(R, 1)
  ji = ji_ref[0].reshape(R, 1)
  oh = ((gi == cls) | (ai == cls - 8) | (ji == cls - 16)).astype(f32)
  accT = accT + _nt(ctrT_ref[...], oh)

  t = jnp.tanh(accT)
  ssum = jnp.sum(t * t, axis=0, keepdims=True)
  norm = jnp.maximum(jnp.sqrt(ssum), 1e-12)
  out_ref[...] = t / norm


def _full(shape):
  return pl.BlockSpec(shape, lambda i: tuple(0 for _ in shape))


def kernel(uid, user_gender, user_age, user_job, uid_table, gender_table,
           age_table, job_table, Wu, bu, Wg, bg, Wa, ba, Wj, bj, Wc, bc):
  uid_rows = _sc_gather(uid_table, jnp.asarray(uid, jnp.int32))

  gi = jnp.asarray(user_gender, jnp.int32).reshape(G, 1, R)
  ai = jnp.asarray(user_age, jnp.int32).reshape(G, 1, R)
  ji = jnp.asarray(user_job, jnp.int32).reshape(G, 1, R)

  idx_spec = pl.BlockSpec((1, 1, R), lambda i: (i, 0, 0))
  out = pl.pallas_call(
      _tc_body,
      grid=(G,),
      in_specs=[
          pl.BlockSpec((R, D), lambda i: (i, 0)),
          idx_spec,
          idx_spec,
          idx_spec,
          _full((GENDER_N, DH)),
          _full((AGE_N, DH)),
          _full((JOB_N, DH)),
          _full((D, D)),
          _full((1, D)),
          _full((DH, D)),
          _full((1, D)),
          _full((DH, D)),
          _full((1, D)),
          _full((DH, D)),
          _full((1, D)),
          _full((OUT, 4 * D)),
          _full((1, OUT)),
      ],
      out_specs=pl.BlockSpec((OUT, R), lambda i: (0, i)),
      out_shape=jax.ShapeDtypeStruct((OUT, B), jnp.float32),
      scratch_shapes=[pltpu.VMEM((OUT, NCLS), jnp.float32)],
  )(uid_rows, gi, ai, ji, gender_table, age_table, job_table,
    Wu, bu.reshape(1, D), Wg, bg.reshape(1, D), Wa, ba.reshape(1, D),
    Wj, bj.reshape(1, D), Wc.T, bc.reshape(1, OUT))
  return out.T


# speedup vs baseline: 3.7242x; 1.0304x over previous
import functools

import jax
import jax.numpy as jnp
from jax import lax
from jax.experimental import pallas as pl
from jax.experimental.pallas import tpu as pltpu
from jax.experimental.pallas import tpu_sc as plsc

B = 4096
D = 128
DH = 64
OUT = 200
R = 2048
G = B // R
NCLS = 40
GENDER_N = 2
AGE_N = 7
JOB_N = 21


def _sc_gather(table, idx):
  info = plsc.get_sparse_core_info()
  nc, ns = info.num_cores, info.num_subcores
  nw = nc * ns
  b_per_w = B // nw
  mesh = plsc.VectorSubcoreMesh(core_axis_name="c", subcore_axis_name="s")

  @functools.partial(
      pl.kernel,
      mesh=mesh,
      out_type=jax.ShapeDtypeStruct((B, D), jnp.float32),
      scratch_types=[
          pltpu.VMEM((b_per_w,), jnp.int32),
          pltpu.VMEM((b_per_w, D), jnp.float32),
          pltpu.SemaphoreType.DMA,
      ],
  )
  def gather_kernel(table_hbm, idx_hbm, out_hbm, idx_v, rows_v, sem):
    wid = lax.axis_index("s") * nc + lax.axis_index("c")
    base = wid * b_per_w
    pltpu.sync_copy(idx_hbm.at[pl.ds(base, b_per_w)], idx_v)
    pltpu.async_copy(table_hbm.at[idx_v], rows_v, sem).wait()
    pltpu.sync_copy(rows_v, out_hbm.at[pl.ds(base, b_per_w)])

  return gather_kernel(table, idx)


def _nt(a, b):
  return lax.dot_general(a, b, (((1,), (1,)), ((), ())),
                         preferred_element_type=jnp.float32)


def _tc_body(uid_rows_ref, gi_ref, ai_ref, ji_ref, g_tab_ref, a_tab_ref,
             j_tab_ref, Wu_ref, bu_ref, Wg_ref, bg_ref, Wa_ref, ba_ref,
             Wj_ref, bj_ref, WcT_ref, bc_ref, out_ref, ctrT_ref):
  f32 = jnp.float32
  i = pl.program_id(0)

  @pl.when(i == 0)
  def _():
    ctrT_ref[...] = jnp.zeros((OUT, NCLS), f32)
    gt = jnp.maximum(
        jnp.dot(g_tab_ref[...], Wg_ref[...], preferred_element_type=f32)
        + bg_ref[...], 0.0)
    at = jnp.maximum(
        jnp.dot(a_tab_ref[...], Wa_ref[...], preferred_element_type=f32)
        + ba_ref[...], 0.0)
    jt = jnp.maximum(
        jnp.dot(j_tab_ref[...], Wj_ref[...], preferred_element_type=f32)
        + bj_ref[...], 0.0)
    bcT = bc_ref[...].reshape(OUT, 1)
    ctrT_ref[:, 0:2] = _nt(WcT_ref[:, D:2 * D], gt) + bcT
    ctrT_ref[:, 8:15] = _nt(WcT_ref[:, 2 * D:3 * D], at)
    ctrT_ref[:, 16:37] = _nt(WcT_ref[:, 3 * D:4 * D], jt)

  u_fc = jnp.maximum(
      jnp.dot(uid_rows_ref[...], Wu_ref[...], preferred_element_type=f32)
      + bu_ref[...], 0.0)
  accT = _nt(WcT_ref[:, 0:D], u_fc)

  cls = lax.broadcasted_iota(jnp.int32, (R, NCLS), 1)
  gi = gi_ref[0].reshape(R, 1)
  ai = ai_ref[0].reshape(R, 1)
  ji = ji_ref[0].reshape(R, 1)
  oh = ((gi == cls) | (ai == cls - 8) | (ji == cls - 16)).astype(f32)
  accT = accT + _nt(ctrT_ref[...], oh)

  t = jnp.tanh(accT)
  ssum = jnp.sum(t * t, axis=0, keepdims=True)
  norm = jnp.maximum(jnp.sqrt(ssum), 1e-12)
  out_ref[...] = t / norm


def _full(shape):
  return pl.BlockSpec(shape, lambda i: tuple(0 for _ in shape))


def kernel(uid, user_gender, user_age, user_job, uid_table, gender_table,
           age_table, job_table, Wu, bu, Wg, bg, Wa, ba, Wj, bj, Wc, bc):
  uid_rows = _sc_gather(uid_table, jnp.asarray(uid, jnp.int32))

  gi = jnp.asarray(user_gender, jnp.int32).reshape(G, 1, R)
  ai = jnp.asarray(user_age, jnp.int32).reshape(G, 1, R)
  ji = jnp.asarray(user_job, jnp.int32).reshape(G, 1, R)

  idx_spec = pl.BlockSpec((1, 1, R), lambda i: (i, 0, 0))
  out = pl.pallas_call(
      _tc_body,
      grid=(G,),
      in_specs=[
          pl.BlockSpec((R, D), lambda i: (i, 0)),
          idx_spec,
          idx_spec,
          idx_spec,
          _full((GENDER_N, DH)),
          _full((AGE_N, DH)),
          _full((JOB_N, DH)),
          _full((D, D)),
          _full((1, D)),
          _full((DH, D)),
          _full((1, D)),
          _full((DH, D)),
          _full((1, D)),
          _full((DH, D)),
          _full((1, D)),
          _full((OUT, 4 * D)),
          _full((1, OUT)),
      ],
      out_specs=pl.BlockSpec((OUT, R), lambda i: (0, i)),
      out_shape=jax.ShapeDtypeStruct((OUT, B), jnp.float32),
      scratch_shapes=[pltpu.VMEM((OUT, NCLS), jnp.float32)],
  )(uid_rows, gi, ai, ji, gender_table, age_table, job_table,
    Wu, bu.reshape(1, D), Wg, bg.reshape(1, D), Wa, ba.reshape(1, D),
    Wj, bj.reshape(1, D), Wc.T, bc.reshape(1, OUT))
  return out.T


# speedup vs baseline: 3.7384x; 1.0038x over previous
import functools

import jax
import jax.numpy as jnp
from jax import lax
from jax.experimental import pallas as pl
from jax.experimental.pallas import tpu as pltpu
from jax.experimental.pallas import tpu_sc as plsc

B = 4096
D = 128
DH = 64
OUT = 200
R = 4096
G = B // R
NCLS = 40
GENDER_N = 2
AGE_N = 7
JOB_N = 21


def _sc_gather(table, idx):
  info = plsc.get_sparse_core_info()
  nc, ns = info.num_cores, info.num_subcores
  nw = nc * ns
  b_per_w = B // nw
  mesh = plsc.VectorSubcoreMesh(core_axis_name="c", subcore_axis_name="s")

  @functools.partial(
      pl.kernel,
      mesh=mesh,
      out_type=jax.ShapeDtypeStruct((B, D), jnp.float32),
      scratch_types=[
          pltpu.VMEM((b_per_w,), jnp.int32),
          pltpu.VMEM((b_per_w, D), jnp.float32),
          pltpu.SemaphoreType.DMA,
      ],
  )
  def gather_kernel(table_hbm, idx_hbm, out_hbm, idx_v, rows_v, sem):
    wid = lax.axis_index("s") * nc + lax.axis_index("c")
    base = wid * b_per_w
    pltpu.sync_copy(idx_hbm.at[pl.ds(base, b_per_w)], idx_v)
    pltpu.async_copy(table_hbm.at[idx_v], rows_v, sem).wait()
    pltpu.sync_copy(rows_v, out_hbm.at[pl.ds(base, b_per_w)])

  return gather_kernel(table, idx)


def _nt(a, b):
  return lax.dot_general(a, b, (((1,), (1,)), ((), ())),
                         preferred_element_type=jnp.float32)


def _tc_body(uid_rows_ref, gi_ref, ai_ref, ji_ref, g_tab_ref, a_tab_ref,
             j_tab_ref, Wu_ref, bu_ref, Wg_ref, bg_ref, Wa_ref, ba_ref,
             Wj_ref, bj_ref, WcT_ref, bc_ref, out_ref, ctrT_ref):
  f32 = jnp.float32
  i = pl.program_id(0)

  @pl.when(i == 0)
  def _():
    ctrT_ref[...] = jnp.zeros((OUT, NCLS), f32)
    gt = jnp.maximum(
        jnp.dot(g_tab_ref[...], Wg_ref[...], preferred_element_type=f32)
        + bg_ref[...], 0.0)
    at = jnp.maximum(
        jnp.dot(a_tab_ref[...], Wa_ref[...], preferred_element_type=f32)
        + ba_ref[...], 0.0)
    jt = jnp.maximum(
        jnp.dot(j_tab_ref[...], Wj_ref[...], preferred_element_type=f32)
        + bj_ref[...], 0.0)
    bcT = bc_ref[...].reshape(OUT, 1)
    ctrT_ref[:, 0:2] = _nt(WcT_ref[:, D:2 * D], gt) + bcT
    ctrT_ref[:, 8:15] = _nt(WcT_ref[:, 2 * D:3 * D], at)
    ctrT_ref[:, 16:37] = _nt(WcT_ref[:, 3 * D:4 * D], jt)

  u_fc = jnp.maximum(
      jnp.dot(uid_rows_ref[...], Wu_ref[...], preferred_element_type=f32)
      + bu_ref[...], 0.0)
  accT = _nt(WcT_ref[:, 0:D], u_fc)

  cls = lax.broadcasted_iota(jnp.int32, (R, NCLS), 1)
  gi = gi_ref[0].reshape(R, 1)
  ai = ai_ref[0].reshape(R, 1)
  ji = ji_ref[0].reshape(R, 1)
  oh = ((gi == cls) | (ai == cls - 8) | (ji == cls - 16)).astype(f32)
  accT = accT + _nt(ctrT_ref[...], oh)

  t = jnp.tanh(accT)
  ssum = jnp.sum(t * t, axis=0, keepdims=True)
  norm = jnp.maximum(jnp.sqrt(ssum), 1e-12)
  out_ref[...] = t / norm


def _full(shape):
  return pl.BlockSpec(shape, lambda i: tuple(0 for _ in shape))


def kernel(uid, user_gender, user_age, user_job, uid_table, gender_table,
           age_table, job_table, Wu, bu, Wg, bg, Wa, ba, Wj, bj, Wc, bc):
  uid_rows = _sc_gather(uid_table, jnp.asarray(uid, jnp.int32))

  gi = jnp.asarray(user_gender, jnp.int32).reshape(G, 1, R)
  ai = jnp.asarray(user_age, jnp.int32).reshape(G, 1, R)
  ji = jnp.asarray(user_job, jnp.int32).reshape(G, 1, R)

  idx_spec = pl.BlockSpec((1, 1, R), lambda i: (i, 0, 0))
  out = pl.pallas_call(
      _tc_body,
      grid=(G,),
      in_specs=[
          pl.BlockSpec((R, D), lambda i: (i, 0)),
          idx_spec,
          idx_spec,
          idx_spec,
          _full((GENDER_N, DH)),
          _full((AGE_N, DH)),
          _full((JOB_N, DH)),
          _full((D, D)),
          _full((1, D)),
          _full((DH, D)),
          _full((1, D)),
          _full((DH, D)),
          _full((1, D)),
          _full((DH, D)),
          _full((1, D)),
          _full((OUT, 4 * D)),
          _full((1, OUT)),
      ],
      out_specs=pl.BlockSpec((OUT, R), lambda i: (0, i)),
      out_shape=jax.ShapeDtypeStruct((OUT, B), jnp.float32),
      scratch_shapes=[pltpu.VMEM((OUT, NCLS), jnp.float32)],
  )(uid_rows, gi, ai, ji, gender_table, age_table, job_table,
    Wu, bu.reshape(1, D), Wg, bg.reshape(1, D), Wa, ba.reshape(1, D),
    Wj, bj.reshape(1, D), Wc.T, bc.reshape(1, OUT))
  return out.T


# speedup vs baseline: 3.7608x; 1.0060x over previous
import functools

import jax
import jax.numpy as jnp
from jax import lax
from jax.experimental import pallas as pl
from jax.experimental.pallas import tpu as pltpu
from jax.experimental.pallas import tpu_sc as plsc

B = 4096
D = 128
DH = 64
OUT = 200
R = 2048
G = B // R
NCLS = 40
GENDER_N = 2
AGE_N = 7
JOB_N = 21


def _sc_gather(table, idx):
  info = plsc.get_sparse_core_info()
  nc, ns = info.num_cores, info.num_subcores
  nw = nc * ns
  b_per_w = B // nw
  mesh = plsc.VectorSubcoreMesh(core_axis_name="c", subcore_axis_name="s")

  @functools.partial(
      pl.kernel,
      mesh=mesh,
      out_type=jax.ShapeDtypeStruct((B, D), jnp.float32),
      scratch_types=[
          pltpu.VMEM((b_per_w,), jnp.int32),
          pltpu.VMEM((b_per_w, D), jnp.float32),
          pltpu.SemaphoreType.DMA,
      ],
  )
  def gather_kernel(table_hbm, idx_hbm, out_hbm, idx_v, rows_v, sem):
    wid = lax.axis_index("s") * nc + lax.axis_index("c")
    base = wid * b_per_w
    pltpu.sync_copy(idx_hbm.at[pl.ds(base, b_per_w)], idx_v)
    pltpu.async_copy(table_hbm.at[idx_v], rows_v, sem).wait()
    pltpu.sync_copy(rows_v, out_hbm.at[pl.ds(base, b_per_w)])

  return gather_kernel(table, idx)


def _nt(a, b):
  return lax.dot_general(a, b, (((1,), (1,)), ((), ())),
                         preferred_element_type=jnp.float32)


def _tc_body(uid_rows_ref, gi_ref, ai_ref, ji_ref, g_tab_ref, a_tab_ref,
             j_tab_ref, Wu_ref, bu_ref, Wg_ref, bg_ref, Wa_ref, ba_ref,
             Wj_ref, bj_ref, WcT_ref, bc_ref, out_ref, ctrT_ref):
  f32 = jnp.float32
  i = pl.program_id(0)

  @pl.when(i == 0)
  def _():
    ctrT_ref[...] = jnp.zeros((OUT, NCLS), f32)
    gt = jnp.maximum(
        jnp.dot(g_tab_ref[...], Wg_ref[...], preferred_element_type=f32)
        + bg_ref[...], 0.0)
    at = jnp.maximum(
        jnp.dot(a_tab_ref[...], Wa_ref[...], preferred_element_type=f32)
        + ba_ref[...], 0.0)
    jt = jnp.maximum(
        jnp.dot(j_tab_ref[...], Wj_ref[...], preferred_element_type=f32)
        + bj_ref[...], 0.0)
    bcT = bc_ref[...].reshape(OUT, 1)
    ctrT_ref[:, 0:2] = _nt(WcT_ref[:, D:2 * D], gt) + bcT
    ctrT_ref[:, 8:15] = _nt(WcT_ref[:, 2 * D:3 * D], at)
    ctrT_ref[:, 16:37] = _nt(WcT_ref[:, 3 * D:4 * D], jt)

  u_fc = jnp.maximum(
      jnp.dot(uid_rows_ref[...], Wu_ref[...], preferred_element_type=f32)
      + bu_ref[...], 0.0)
  accT = _nt(WcT_ref[:, 0:D], u_fc)

  cls = lax.broadcasted_iota(jnp.int32, (R, NCLS), 1)
  gi = gi_ref[0].reshape(R, 1)
  ai = ai_ref[0].reshape(R, 1)
  ji = ji_ref[0].reshape(R, 1)
  oh = ((gi == cls) | (ai == cls - 8) | (ji == cls - 16)).astype(f32)
  accT = accT + _nt(ctrT_ref[...], oh)

  t = jnp.tanh(accT)
  ssum = jnp.sum(t * t, axis=0, keepdims=True)
  norm = jnp.maximum(jnp.sqrt(ssum), 1e-12)
  out_ref[...] = t / norm


def _full(shape):
  return pl.BlockSpec(shape, lambda i: tuple(0 for _ in shape))


def kernel(uid, user_gender, user_age, user_job, uid_table, gender_table,
           age_table, job_table, Wu, bu, Wg, bg, Wa, ba, Wj, bj, Wc, bc):
  uid_rows = _sc_gather(uid_table, jnp.asarray(uid, jnp.int32))

  gi = jnp.asarray(user_gender, jnp.int32).reshape(G, 1, R)
  ai = jnp.asarray(user_age, jnp.int32).reshape(G, 1, R)
  ji = jnp.asarray(user_job, jnp.int32).reshape(G, 1, R)

  idx_spec = pl.BlockSpec((1, 1, R), lambda i: (i, 0, 0))
  out = pl.pallas_call(
      _tc_body,
      grid=(G,),
      in_specs=[
          pl.BlockSpec((R, D), lambda i: (i, 0)),
          idx_spec,
          idx_spec,
          idx_spec,
          _full((GENDER_N, DH)),
          _full((AGE_N, DH)),
          _full((JOB_N, DH)),
          _full((D, D)),
          _full((1, D)),
          _full((DH, D)),
          _full((1, D)),
          _full((DH, D)),
          _full((1, D)),
          _full((DH, D)),
          _full((1, D)),
          _full((OUT, 4 * D)),
          _full((1, OUT)),
      ],
      out_specs=pl.BlockSpec((OUT, R), lambda i: (0, i)),
      out_shape=jax.ShapeDtypeStruct((OUT, B), jnp.float32),
      scratch_shapes=[pltpu.VMEM((OUT, NCLS), jnp.float32)],
  )(uid_rows, gi, ai, ji, gender_table, age_table, job_table,
    Wu, bu.reshape(1, D), Wg, bg.reshape(1, D), Wa, ba.reshape(1, D),
    Wj, bj.reshape(1, D), Wc.T, bc.reshape(1, OUT))
  return out.T
